# Initial kernel scaffold; baseline (speedup 1.0000x reference)
import jax
import jax.numpy as jnp
from jax.experimental import pallas as pl


def kernel(x, labels, class_means, global_mean):
    raise NotImplementedError("write your pallas kernel here")


# speedup vs baseline: 1.2381x; 1.2381x over previous
import jax
import jax.numpy as jnp
from jax import lax
from jax.experimental import pallas as pl
from jax.experimental.pallas import tpu as pltpu
from jax.experimental.pallas import tpu_sc as plsc

_NC = 2
_NS = 16
_NW = _NC * _NS
_L = 16

_N = 16384
_F = 2
_FLAT = _N * _F
_ROWS_PER = _N // _NW
_FLAT_PER = _FLAT // _NW
_VECS = _FLAT_PER // _L


def _body(x_hbm, labels_hbm, cm_hbm, out_hbm, x_v, lab_v, cm_v, out_v):
    wid = lax.axis_index("s") * _NC + lax.axis_index("c")
    rbase = wid * _ROWS_PER
    fbase = wid * _FLAT_PER

    pltpu.sync_copy(x_hbm.at[pl.ds(fbase, _FLAT_PER)], x_v)
    pltpu.sync_copy(labels_hbm.at[pl.ds(rbase, _ROWS_PER)], lab_v)
    pltpu.sync_copy(cm_hbm, cm_v)

    iota = lax.iota(jnp.int32, _L)
    half = iota >> 1
    feat = iota & 1

    for i in range(_VECS):
        pos = i * _L
        ridx = (pos >> 1) + half
        lab = plsc.load_gather(lab_v, [ridx])
        g = plsc.load_gather(cm_v, [lab * 2 + feat])
        out_v[pl.ds(pos, _L)] = x_v[pl.ds(pos, _L)] - g

    pltpu.sync_copy(out_v, out_hbm.at[pl.ds(fbase, _FLAT_PER)])


_sc_call = pl.kernel(
    _body,
    out_type=jax.ShapeDtypeStruct((_FLAT,), jnp.float32),
    mesh=plsc.VectorSubcoreMesh(core_axis_name="c", subcore_axis_name="s"),
    compiler_params=pltpu.CompilerParams(needs_layout_passes=False),
    scratch_types=[
        pltpu.VMEM((_FLAT_PER,), jnp.float32),
        pltpu.VMEM((_ROWS_PER,), jnp.int32),
        pltpu.VMEM((_L,), jnp.float32),
        pltpu.VMEM((_FLAT_PER,), jnp.float32),
    ],
)


@jax.jit
def kernel(x, labels, class_means, global_mean):
    del global_mean
    x_flat = x.reshape(_FLAT)
    lab = labels.astype(jnp.int32)
    cm_pad = jnp.pad(class_means.reshape(_F * 3), (0, _L - _F * 3))
    out_flat = _sc_call(x_flat, lab, cm_pad)
    return out_flat.reshape(_N, _F)
